# Initial kernel scaffold; baseline (speedup 1.0000x reference)
import jax
import jax.numpy as jnp
from jax import lax
from jax.experimental import pallas as pl
from jax.experimental.pallas import tpu as pltpu
from jax.experimental.pallas import tpu_sc as plsc

_N = 10000
_D = 128
_H = 64
_NS = 16
_L = 16
_CH = 128
_RPT = _N // _NS


def _make_spmm(cpt):
    mesh = plsc.VectorSubcoreMesh(core_axis_name="c", subcore_axis_name="s")

    def body(src_hbm, dst_hbm, w_hbm, x0_hbm, x1_hbm, o0_hbm, o1_hbm,
             idx_s, idx_d, w_v, rows, zbuf, acc, sem):
        c = lax.axis_index("c")
        s = lax.axis_index("s")

        def run_half(x_hbm, o_hbm):
            def zrow(i, carry):
                for j in range(_H // _L):
                    zbuf[i, pl.ds(j * _L, _L)] = jnp.zeros((_L,), jnp.float32)
                return carry
            lax.fori_loop(0, _RPT, zrow, 0)
            row0 = s * _RPT
            pltpu.sync_copy(zbuf, acc.at[pl.ds(row0, _RPT)])
            plsc.subcore_barrier()

            def chunk(k, carry):
                base = (s * cpt + k) * _CH
                pltpu.sync_copy(src_hbm.at[pl.ds(base, _CH)], idx_s)
                pltpu.sync_copy(dst_hbm.at[pl.ds(base, _CH)], idx_d)
                pltpu.sync_copy(w_hbm.at[pl.ds(base, _CH)], w_v)
                pltpu.async_copy(x_hbm.at[idx_s], rows, sem).wait()

                def scale(e, c2):
                    wb = plsc.load_gather(w_v, [jnp.full((_L,), e, jnp.int32)])
                    for j in range(_H // _L):
                        sl = pl.ds(j * _L, _L)
                        rows[e, sl] = rows[e, sl] * wb
                    return c2
                lax.fori_loop(0, _CH, scale, 0)
                pltpu.sync_copy(rows, acc.at[idx_d], add=True)
                return carry
            lax.fori_loop(0, cpt, chunk, 0)
            plsc.subcore_barrier()
            pltpu.sync_copy(acc.at[pl.ds(row0, _RPT)],
                            o_hbm.at[pl.ds(row0, _RPT)])

        @pl.when(c == 0)
        def _():
            run_half(x0_hbm, o0_hbm)

        @pl.when(c == 1)
        def _():
            run_half(x1_hbm, o1_hbm)

    return pl.kernel(
        body,
        out_type=(jax.ShapeDtypeStruct((_N, _H), jnp.float32),
                  jax.ShapeDtypeStruct((_N, _H), jnp.float32)),
        mesh=mesh,
        scratch_types=[
            pltpu.VMEM((_CH,), jnp.int32),
            pltpu.VMEM((_CH,), jnp.int32),
            pltpu.VMEM((_CH,), jnp.float32),
            pltpu.VMEM((_CH, _H), jnp.float32),
            pltpu.VMEM((_RPT, _H), jnp.float32),
            pltpu.VMEM_SHARED((_N, _H), jnp.float32),
            pltpu.SemaphoreType.DMA,
        ],
    )


def _linear(h0, h1, wa, wb, b2):
    bm = 1000

    def mm(h0_ref, h1_ref, wa_ref, wb_ref, b_ref, o_ref):
        o_ref[...] = (
            jnp.dot(h0_ref[...], wa_ref[...], preferred_element_type=jnp.float32)
            + jnp.dot(h1_ref[...], wb_ref[...], preferred_element_type=jnp.float32)
            + b_ref[...])

    return pl.pallas_call(
        mm,
        grid=(_N // bm,),
        in_specs=[
            pl.BlockSpec((bm, _H), lambda i: (i, 0)),
            pl.BlockSpec((bm, _H), lambda i: (i, 0)),
            pl.BlockSpec((_H, _D), lambda i: (0, 0)),
            pl.BlockSpec((_H, _D), lambda i: (0, 0)),
            pl.BlockSpec((1, _D), lambda i: (0, 0)),
        ],
        out_specs=pl.BlockSpec((bm, _D), lambda i: (i, 0)),
        out_shape=jax.ShapeDtypeStruct((_N, _D), jnp.float32),
    )(h0, h1, wa, wb, b2)


def kernel(x, edge_index, edge_weight, W, b):
    e = edge_index.shape[1]
    src = edge_index[0].astype(jnp.int32)
    dst = edge_index[1].astype(jnp.int32)
    w = edge_weight.astype(jnp.float32)
    cpt = -(-e // (_NS * _CH))
    ep = cpt * _NS * _CH
    if ep != e:
        src = jnp.pad(src, (0, ep - e))
        dst = jnp.pad(dst, (0, ep - e))
        w = jnp.pad(w, (0, ep - e))
    x0 = x[:, :_H]
    x1 = x[:, _H:]
    spmm = _make_spmm(cpt)
    h10, h11 = spmm(src, dst, w, x0, x1)
    h20, h21 = spmm(src, dst, w, h10, h11)
    return _linear(h20, h21, W[:_H, :], W[_H:, :], b.reshape(1, _D))


# speedup vs baseline: 2.1248x; 2.1248x over previous
import jax
import jax.numpy as jnp
from jax import lax
from jax.experimental import pallas as pl
from jax.experimental.pallas import tpu as pltpu
from jax.experimental.pallas import tpu_sc as plsc

_N = 10000
_D = 128
_H = 64
_NS = 16
_L = 16
_CH = 128
_RPT = 632
_NP = _RPT * _NS


def _make_spmm(cpt):
    mesh = plsc.VectorSubcoreMesh(core_axis_name="c", subcore_axis_name="s")

    def body(src_hbm, dst_hbm, w_hbm, x0_hbm, x1_hbm, o0_hbm, o1_hbm,
             idx_s, idx_d, w_v, rows, zbuf, acc, sem):
        c = lax.axis_index("c")
        s = lax.axis_index("s")

        def run_half(x_hbm, o_hbm):
            def zrow(i, carry):
                for j in range(_H // _L):
                    zbuf[i, pl.ds(j * _L, _L)] = jnp.zeros((_L,), jnp.float32)
                return carry
            lax.fori_loop(0, _RPT, zrow, 0)
            row0 = s * _RPT
            pltpu.sync_copy(zbuf, acc.at[pl.ds(row0, _RPT)])
            plsc.subcore_barrier()

            def chunk(k, carry):
                base = (s * cpt + k) * _CH
                pltpu.sync_copy(src_hbm.at[pl.ds(base, _CH)], idx_s)
                pltpu.sync_copy(dst_hbm.at[pl.ds(base, _CH)], idx_d)
                pltpu.sync_copy(w_hbm.at[pl.ds(base, _CH)], w_v)
                pltpu.async_copy(x_hbm.at[idx_s], rows, sem).wait()

                def scale(g, c2):
                    w16 = w_v[pl.ds(g * _L, _L)]
                    for e16 in range(_L):
                        wb = jnp.broadcast_to(w16[e16], (_L,))
                        e = g * _L + e16
                        for j in range(_H // _L):
                            sl = pl.ds(j * _L, _L)
                            rows[e, sl] = rows[e, sl] * wb
                    return c2
                lax.fori_loop(0, _CH // _L, scale, 0)
                pltpu.sync_copy(rows, acc.at[idx_d], add=True)
                return carry
            lax.fori_loop(0, cpt, chunk, 0)
            plsc.subcore_barrier()
            pltpu.sync_copy(acc.at[pl.ds(row0, _RPT)],
                            o_hbm.at[pl.ds(row0, _RPT)])

        @pl.when(c == 0)
        def _():
            run_half(x0_hbm, o0_hbm)

        @pl.when(c == 1)
        def _():
            run_half(x1_hbm, o1_hbm)

    return pl.kernel(
        body,
        out_type=(jax.ShapeDtypeStruct((_NP, _H), jnp.float32),
                  jax.ShapeDtypeStruct((_NP, _H), jnp.float32)),
        mesh=mesh,
        scratch_types=[
            pltpu.VMEM((_CH,), jnp.int32),
            pltpu.VMEM((_CH,), jnp.int32),
            pltpu.VMEM((_CH,), jnp.float32),
            pltpu.VMEM((_CH, _H), jnp.float32),
            pltpu.VMEM((_RPT, _H), jnp.float32),
            pltpu.VMEM_SHARED((_NP, _H), jnp.float32),
            pltpu.SemaphoreType.DMA,
        ],
        compiler_params=pltpu.CompilerParams(use_tc_tiling_on_sc=False),
    )


def _linear(h0, h1, wa, wb, b2):
    bm = 1000

    def mm(h0_ref, h1_ref, wa_ref, wb_ref, b_ref, o_ref):
        o_ref[...] = (
            jnp.dot(h0_ref[...], wa_ref[...], preferred_element_type=jnp.float32)
            + jnp.dot(h1_ref[...], wb_ref[...], preferred_element_type=jnp.float32)
            + b_ref[...])

    return pl.pallas_call(
        mm,
        grid=(_N // bm,),
        in_specs=[
            pl.BlockSpec((bm, _H), lambda i: (i, 0)),
            pl.BlockSpec((bm, _H), lambda i: (i, 0)),
            pl.BlockSpec((_H, _D), lambda i: (0, 0)),
            pl.BlockSpec((_H, _D), lambda i: (0, 0)),
            pl.BlockSpec((1, _D), lambda i: (0, 0)),
        ],
        out_specs=pl.BlockSpec((bm, _D), lambda i: (i, 0)),
        out_shape=jax.ShapeDtypeStruct((_N, _D), jnp.float32),
    )(h0, h1, wa, wb, b2)


def kernel(x, edge_index, edge_weight, W, b):
    e = edge_index.shape[1]
    src = edge_index[0].astype(jnp.int32)
    dst = edge_index[1].astype(jnp.int32)
    w = edge_weight.astype(jnp.float32)
    cpt = -(-e // (_NS * _CH))
    ep = cpt * _NS * _CH
    if ep != e:
        src = jnp.pad(src, (0, ep - e))
        dst = jnp.pad(dst, (0, ep - e))
        w = jnp.pad(w, (0, ep - e))
    x0 = x[:, :_H]
    x1 = x[:, _H:]
    spmm = _make_spmm(cpt)
    h10, h11 = spmm(src, dst, w, x0, x1)
    h20, h21 = spmm(src, dst, w, h10, h11)
    return _linear(h20[:_N], h21[:_N], W[:_H, :], W[_H:, :],
                   b.reshape(1, _D))


# speedup vs baseline: 3.0042x; 1.4138x over previous
import jax
import jax.numpy as jnp
from jax import lax
from jax.experimental import pallas as pl
from jax.experimental.pallas import tpu as pltpu
from jax.experimental.pallas import tpu_sc as plsc

_N = 10000
_D = 128
_H = 64
_NS = 16
_L = 16
_CH = 128
_RPT = 632
_NP = _RPT * _NS


_G = 8


def _make_spmm(cpt):
    mesh = plsc.VectorSubcoreMesh(core_axis_name="c", subcore_axis_name="s")

    def body(src_hbm, dst_hbm, w_hbm, x0_hbm, x1_hbm, o0_hbm, o1_hbm,
             idx_s, idx_d, w_v, rows0, rows1, zbuf, acc,
             sem_i, sem_g0, sem_g1, sem_s0, sem_s1):
        c = lax.axis_index("c")
        s = lax.axis_index("s")
        rows = (rows0, rows1)
        sem_g = (sem_g0, sem_g1)
        sem_s = (sem_s0, sem_s1)

        def scale_chunk(buf, j):
            def scale(g, c2):
                w16 = w_v[j, pl.ds(g * _L, _L)]
                for e16 in range(_L):
                    wb = jnp.broadcast_to(w16[e16], (_L,))
                    e = g * _L + e16
                    for q in range(_H // _L):
                        sl = pl.ds(q * _L, _L)
                        buf[e, sl] = buf[e, sl] * wb
                return c2
            lax.fori_loop(0, _CH // _L, scale, 0)

        def run_half(x_hbm, o_hbm):
            def zrow(i, carry):
                for j in range(_H // _L):
                    zbuf[i, pl.ds(j * _L, _L)] = jnp.zeros((_L,), jnp.float32)
                return carry
            lax.fori_loop(0, _RPT, zrow, 0)
            row0 = s * _RPT
            pltpu.sync_copy(zbuf, acc.at[pl.ds(row0, _RPT)])
            plsc.subcore_barrier()

            def group(gk, carry):
                crow = s * cpt + gk * _G
                di = pltpu.async_copy(src_hbm.at[pl.ds(crow, _G)], idx_s, sem_i)
                dd = pltpu.async_copy(dst_hbm.at[pl.ds(crow, _G)], idx_d, sem_i)
                dw = pltpu.async_copy(w_hbm.at[pl.ds(crow, _G)], w_v, sem_i)
                di.wait()
                dd.wait()
                dw.wait()
                gd = [
                    pltpu.async_copy(x_hbm.at[idx_s.at[0]], rows[0], sem_g[0]),
                    pltpu.async_copy(x_hbm.at[idx_s.at[1]], rows[1], sem_g[1]),
                ]
                sd = [None, None]
                for j in range(_G):
                    b = j & 1
                    ob = b ^ 1
                    if 1 <= j < _G - 1:
                        sd[ob].wait()
                        gd[ob] = pltpu.async_copy(
                            x_hbm.at[idx_s.at[j + 1]], rows[ob], sem_g[ob])
                    gd[b].wait()
                    scale_chunk(rows[b], j)
                    sd[b] = pltpu.async_copy(
                        rows[b], acc.at[idx_d.at[j]], sem_s[b], add=True)
                sd[0].wait()
                sd[1].wait()
                return carry
            lax.fori_loop(0, cpt // _G, group, 0)
            plsc.subcore_barrier()
            pltpu.sync_copy(acc.at[pl.ds(row0, _RPT)],
                            o_hbm.at[pl.ds(row0, _RPT)])

        @pl.when(c == 0)
        def _():
            run_half(x0_hbm, o0_hbm)

        @pl.when(c == 1)
        def _():
            run_half(x1_hbm, o1_hbm)

    return pl.kernel(
        body,
        out_type=(jax.ShapeDtypeStruct((_NP, _H), jnp.float32),
                  jax.ShapeDtypeStruct((_NP, _H), jnp.float32)),
        mesh=mesh,
        scratch_types=[
            pltpu.VMEM((_G, _CH), jnp.int32),
            pltpu.VMEM((_G, _CH), jnp.int32),
            pltpu.VMEM((_G, _CH), jnp.float32),
            pltpu.VMEM((_CH, _H), jnp.float32),
            pltpu.VMEM((_CH, _H), jnp.float32),
            pltpu.VMEM((_RPT, _H), jnp.float32),
            pltpu.VMEM_SHARED((_NP, _H), jnp.float32),
            pltpu.SemaphoreType.DMA,
            pltpu.SemaphoreType.DMA,
            pltpu.SemaphoreType.DMA,
            pltpu.SemaphoreType.DMA,
            pltpu.SemaphoreType.DMA,
        ],
        compiler_params=pltpu.CompilerParams(use_tc_tiling_on_sc=False),
    )


def _linear(h0, h1, wa, wb, b2):
    bm = 1000

    def mm(h0_ref, h1_ref, wa_ref, wb_ref, b_ref, o_ref):
        o_ref[...] = (
            jnp.dot(h0_ref[...], wa_ref[...], preferred_element_type=jnp.float32)
            + jnp.dot(h1_ref[...], wb_ref[...], preferred_element_type=jnp.float32)
            + b_ref[...])

    return pl.pallas_call(
        mm,
        grid=(_N // bm,),
        in_specs=[
            pl.BlockSpec((bm, _H), lambda i: (i, 0)),
            pl.BlockSpec((bm, _H), lambda i: (i, 0)),
            pl.BlockSpec((_H, _D), lambda i: (0, 0)),
            pl.BlockSpec((_H, _D), lambda i: (0, 0)),
            pl.BlockSpec((1, _D), lambda i: (0, 0)),
        ],
        out_specs=pl.BlockSpec((bm, _D), lambda i: (i, 0)),
        out_shape=jax.ShapeDtypeStruct((_N, _D), jnp.float32),
    )(h0, h1, wa, wb, b2)


def kernel(x, edge_index, edge_weight, W, b):
    e = edge_index.shape[1]
    src = edge_index[0].astype(jnp.int32)
    dst = edge_index[1].astype(jnp.int32)
    w = edge_weight.astype(jnp.float32)
    cpt = -(-e // (_NS * _CH))
    cpt = -(-cpt // _G) * _G
    ep = cpt * _NS * _CH
    if ep != e:
        src = jnp.pad(src, (0, ep - e))
        dst = jnp.pad(dst, (0, ep - e))
        w = jnp.pad(w, (0, ep - e))
    src = src.reshape(-1, _CH)
    dst = dst.reshape(-1, _CH)
    w = w.reshape(-1, _CH)
    x0 = x[:, :_H]
    x1 = x[:, _H:]
    spmm = _make_spmm(cpt)
    h10, h11 = spmm(src, dst, w, x0, x1)
    h20, h21 = spmm(src, dst, w, h10, h11)
    return _linear(h20[:_N], h21[:_N], W[:_H, :], W[_H:, :],
                   b.reshape(1, _D))


# speedup vs baseline: 4.9204x; 1.6379x over previous
import jax
import jax.numpy as jnp
from jax import lax
from jax.experimental import pallas as pl
from jax.experimental.pallas import tpu as pltpu
from jax.experimental.pallas import tpu_sc as plsc

_N = 10000
_D = 128
_H = 64
_NS = 16
_L = 16
_CH = 128
_RPT = 632
_NP = _RPT * _NS


_G = 32
_NB = 4
_ZR = _RPT // 4


def _make_spmm(cpt):
    mesh = plsc.VectorSubcoreMesh(core_axis_name="c", subcore_axis_name="s")
    nsub = _G // _NB

    def body(src_hbm, dst_hbm, w_hbm, x0_hbm, x1_hbm, o0_hbm, o1_hbm,
             idx_s, idx_d, w_v, rin0, rin1, rin2, rin3, rout0, rout1,
             zbuf, acc, sem_i, sem_g0, sem_g1, sem_g2, sem_g3,
             sem_s0, sem_s1):
        c = lax.axis_index("c")
        s = lax.axis_index("s")
        rin = (rin0, rin1, rin2, rin3)
        rout = (rout0, rout1)
        sem_g = (sem_g0, sem_g1, sem_g2, sem_g3)
        sem_s = (sem_s0, sem_s1)

        def scale_chunk(src_buf, dst_buf, j):
            @plsc.parallel_loop(0, _CH // _L, unroll=2)
            def scale(g):
                w16 = w_v[j, pl.ds(g * _L, _L)]
                for e16 in range(_L):
                    wb = jnp.broadcast_to(w16[e16], (_L,))
                    e = g * _L + e16
                    for q in range(_H // _L):
                        sl = pl.ds(q * _L, _L)
                        dst_buf[e, sl] = src_buf[e, sl] * wb

        def run_half(x_hbm, o_hbm):
            def gwait(b):
                pltpu.make_async_copy(
                    x_hbm.at[idx_s.at[0]], rin[b], sem_g[b]).wait()

            def swait(o):
                pltpu.make_async_copy(
                    rout[o], acc.at[idx_d.at[0]], sem_s[o]).wait()

            def zrow(i, carry):
                for j in range(_H // _L):
                    zbuf[i, pl.ds(j * _L, _L)] = jnp.zeros((_L,), jnp.float32)
                return carry
            lax.fori_loop(0, _ZR, zrow, 0)
            row0 = s * _RPT
            for t in range(_RPT // _ZR):
                pltpu.sync_copy(zbuf, acc.at[pl.ds(row0 + t * _ZR, _ZR)])
            plsc.subcore_barrier()

            def group(gk, carry):
                crow = s * cpt + gk * _G
                di = pltpu.async_copy(src_hbm.at[pl.ds(crow, _G)], idx_s, sem_i)
                dd = pltpu.async_copy(dst_hbm.at[pl.ds(crow, _G)], idx_d, sem_i)
                dw = pltpu.async_copy(w_hbm.at[pl.ds(crow, _G)], w_v, sem_i)
                di.wait()
                dd.wait()
                dw.wait()
                for b in range(_NB):
                    pltpu.async_copy(x_hbm.at[idx_s.at[b]], rin[b], sem_g[b])

                def sub(q, c2):
                    for jj in range(_NB):
                        j = q * _NB + jj
                        b = jj
                        o = jj & 1
                        gwait(b)
                        if jj < 2:
                            @pl.when(q > 0)
                            def _():
                                swait(o)
                        else:
                            swait(o)
                        scale_chunk(rin[b], rout[o], j)

                        @pl.when(q < nsub - 1)
                        def _():
                            pltpu.async_copy(
                                x_hbm.at[idx_s.at[j + _NB]], rin[b], sem_g[b])
                        pltpu.async_copy(
                            rout[o], acc.at[idx_d.at[j]], sem_s[o], add=True)
                    return c2
                lax.fori_loop(0, nsub, sub, 0)
                swait(0)
                swait(1)
                return carry
            lax.fori_loop(0, cpt // _G, group, 0)
            plsc.subcore_barrier()
            pltpu.sync_copy(acc.at[pl.ds(row0, _RPT)],
                            o_hbm.at[pl.ds(row0, _RPT)])

        @pl.when(c == 0)
        def _():
            run_half(x0_hbm, o0_hbm)

        @pl.when(c == 1)
        def _():
            run_half(x1_hbm, o1_hbm)

    return pl.kernel(
        body,
        out_type=(jax.ShapeDtypeStruct((_NP, _H), jnp.float32),
                  jax.ShapeDtypeStruct((_NP, _H), jnp.float32)),
        mesh=mesh,
        scratch_types=[
            pltpu.VMEM((_G, _CH), jnp.int32),
            pltpu.VMEM((_G, _CH), jnp.int32),
            pltpu.VMEM((_G, _CH), jnp.float32),
            pltpu.VMEM((_CH, _H), jnp.float32),
            pltpu.VMEM((_CH, _H), jnp.float32),
            pltpu.VMEM((_CH, _H), jnp.float32),
            pltpu.VMEM((_CH, _H), jnp.float32),
            pltpu.VMEM((_CH, _H), jnp.float32),
            pltpu.VMEM((_CH, _H), jnp.float32),
            pltpu.VMEM((_ZR, _H), jnp.float32),
            pltpu.VMEM_SHARED((_NP, _H), jnp.float32),
            pltpu.SemaphoreType.DMA,
            pltpu.SemaphoreType.DMA,
            pltpu.SemaphoreType.DMA,
            pltpu.SemaphoreType.DMA,
            pltpu.SemaphoreType.DMA,
            pltpu.SemaphoreType.DMA,
            pltpu.SemaphoreType.DMA,
        ],
        compiler_params=pltpu.CompilerParams(use_tc_tiling_on_sc=False),
    )


def _linear(h0, h1, wa, wb, b2):
    bm = 1000

    def mm(h0_ref, h1_ref, wa_ref, wb_ref, b_ref, o_ref):
        o_ref[...] = (
            jnp.dot(h0_ref[...], wa_ref[...], preferred_element_type=jnp.float32)
            + jnp.dot(h1_ref[...], wb_ref[...], preferred_element_type=jnp.float32)
            + b_ref[...])

    return pl.pallas_call(
        mm,
        grid=(_N // bm,),
        in_specs=[
            pl.BlockSpec((bm, _H), lambda i: (i, 0)),
            pl.BlockSpec((bm, _H), lambda i: (i, 0)),
            pl.BlockSpec((_H, _D), lambda i: (0, 0)),
            pl.BlockSpec((_H, _D), lambda i: (0, 0)),
            pl.BlockSpec((1, _D), lambda i: (0, 0)),
        ],
        out_specs=pl.BlockSpec((bm, _D), lambda i: (i, 0)),
        out_shape=jax.ShapeDtypeStruct((_N, _D), jnp.float32),
    )(h0, h1, wa, wb, b2)


def kernel(x, edge_index, edge_weight, W, b):
    e = edge_index.shape[1]
    src = edge_index[0].astype(jnp.int32)
    dst = edge_index[1].astype(jnp.int32)
    w = edge_weight.astype(jnp.float32)
    cpt = -(-e // (_NS * _CH))
    cpt = -(-cpt // _G) * _G
    assert _RPT % _ZR == 0 and _G % _NB == 0
    ep = cpt * _NS * _CH
    if ep != e:
        src = jnp.pad(src, (0, ep - e))
        dst = jnp.pad(dst, (0, ep - e))
        w = jnp.pad(w, (0, ep - e))
    src = src.reshape(-1, _CH)
    dst = dst.reshape(-1, _CH)
    w = w.reshape(-1, _CH)
    x0 = x[:, :_H]
    x1 = x[:, _H:]
    spmm = _make_spmm(cpt)
    h10, h11 = spmm(src, dst, w, x0, x1)
    h20, h21 = spmm(src, dst, w, h10, h11)
    return _linear(h20[:_N], h21[:_N], W[:_H, :], W[_H:, :],
                   b.reshape(1, _D))
